# Initial kernel scaffold; baseline (speedup 1.0000x reference)
import jax
import jax.numpy as jnp
from jax.experimental import pallas as pl


def kernel(feat_user, feat_movie, edge_src_user, edge_dst_movie, W_self1_m, W_neigh1_m, b1_m, W_self1_u, W_neigh1_u, b1_u, W_self2_m, W_neigh2_m, b2_m, W_self2_u, W_neigh2_u, b2_u):
    raise NotImplementedError("write your pallas kernel here")


# speedup vs baseline: 7.7799x; 7.7799x over previous
import functools

import jax
import jax.numpy as jnp
from jax import lax
from jax.experimental import pallas as pl
from jax.experimental.pallas import tpu as pltpu
from jax.experimental.pallas import tpu_sc as plsc

N_U = 50000
N_M = 50000
E = 1600000
H = 40
OUT = 30

W = 32
CH = 128
NSUB = 16
NCHUNK = E // CH
NROWCHUNK = -(-N_M // CH)
LAST_ROWS = N_M - (NROWCHUNK - 1) * CH

_sc_mesh = plsc.VectorSubcoreMesh(core_axis_name="c", subcore_axis_name="s")


@functools.partial(
    pl.kernel,
    mesh=_sc_mesh,
    compiler_params=pltpu.CompilerParams(use_tc_tiling_on_sc=False),
    out_type=(
        jax.ShapeDtypeStruct((N_M, W), jnp.float32),
        jax.ShapeDtypeStruct((N_U, W), jnp.float32),
    ),
    scratch_types=[
        pltpu.VMEM((CH,), jnp.int32),
        pltpu.VMEM((CH,), jnp.int32),
        pltpu.VMEM((CH, W), jnp.float32),
        pltpu.VMEM_SHARED((N_M, W), jnp.float32),
        pltpu.SemaphoreType.DMA,
    ],
)
def _edge_agg(tab_u, tab_m, src_hbm, dst_hbm, accm_out, accu_out,
              gidx, sidx, rows, acc, sem):
    cid = lax.axis_index("c")
    sid = lax.axis_index("s")

    zv = jnp.zeros((16,), jnp.float32)

    def _zb(r, carry):
        rows[r, pl.ds(0, 16)] = zv
        rows[r, pl.ds(16, 16)] = zv
        return carry

    lax.fori_loop(0, CH, _zb, 0)

    def _zero_chunk(j, carry):
        k = j * NSUB + sid

        @pl.when(k < NROWCHUNK - 1)
        def _():
            pltpu.sync_copy(rows, acc.at[pl.ds(k * CH, CH)])

        @pl.when(k == NROWCHUNK - 1)
        def _():
            pltpu.sync_copy(rows.at[pl.ds(0, LAST_ROWS)],
                            acc.at[pl.ds(k * CH, LAST_ROWS)])

        return carry

    lax.fori_loop(0, -(-NROWCHUNK // NSUB), _zero_chunk, 0)
    plsc.subcore_barrier()

    def _edge_chunk(j, carry):
        k = j * NSUB + sid

        @pl.when(k < NCHUNK)
        def _():
            base = k * CH
            pltpu.sync_copy(src_hbm.at[pl.ds(base, CH)], gidx)
            pltpu.sync_copy(dst_hbm.at[pl.ds(base, CH)], sidx)

            @pl.when(cid == 0)
            def _():
                pltpu.async_copy(tab_u.at[gidx], rows, sem).wait()
                pltpu.sync_copy(rows, acc.at[sidx], add=True)

            @pl.when(cid == 1)
            def _():
                pltpu.async_copy(tab_m.at[sidx], rows, sem).wait()
                pltpu.sync_copy(rows, acc.at[gidx], add=True)

        return carry

    lax.fori_loop(0, -(-NCHUNK // NSUB), _edge_chunk, 0)
    plsc.subcore_barrier()

    def _flush_chunk(j, carry):
        k = j * NSUB + sid

        @pl.when(k < NROWCHUNK - 1)
        def _():
            @pl.when(cid == 0)
            def _():
                pltpu.sync_copy(acc.at[pl.ds(k * CH, CH)],
                                accm_out.at[pl.ds(k * CH, CH)])

            @pl.when(cid == 1)
            def _():
                pltpu.sync_copy(acc.at[pl.ds(k * CH, CH)],
                                accu_out.at[pl.ds(k * CH, CH)])

        @pl.when(k == NROWCHUNK - 1)
        def _():
            @pl.when(cid == 0)
            def _():
                pltpu.sync_copy(acc.at[pl.ds(k * CH, LAST_ROWS)],
                                accm_out.at[pl.ds(k * CH, LAST_ROWS)])

            @pl.when(cid == 1)
            def _():
                pltpu.sync_copy(acc.at[pl.ds(k * CH, LAST_ROWS)],
                                accu_out.at[pl.ds(k * CH, LAST_ROWS)])

        return carry

    lax.fori_loop(0, -(-NROWCHUNK // NSUB), _flush_chunk, 0)


R = 2000
GRID = N_M // R


def _dense1_body(fm, am, fu, au, wsm, wnm, bm, wsu, wnu, bu,
                 wsm2, wnm2, bm2, wsu2, wnu2, bu2,
                 pm_out, pu_out, s2m_out, s2u_out):
    col30 = (lax.broadcasted_iota(jnp.int32, (1, W), 1) == OUT).astype(jnp.float32)

    am_ = am[...]
    rdeg_m = 1.0 / jnp.maximum(am_[:, 20:21], 1.0)
    agg_m = am_[:, :20] * rdeg_m
    h_m = jax.nn.relu(
        jnp.dot(fm[...], wsm[...], preferred_element_type=jnp.float32)
        + jnp.dot(agg_m, wnm[...], preferred_element_type=jnp.float32)
        + bm[...])

    au_ = au[...]
    rdeg_u = 1.0 / jnp.maximum(au_[:, 21:22], 1.0)
    agg_u = au_[:, :21] * rdeg_u
    h_u = jax.nn.relu(
        jnp.dot(fu[...], wsu[...], preferred_element_type=jnp.float32)
        + jnp.dot(agg_u, wnu[...], preferred_element_type=jnp.float32)
        + bu[...])

    pu_out[...] = jnp.dot(h_u, wnm2[...], preferred_element_type=jnp.float32)
    pm_out[...] = jnp.dot(h_m, wnu2[...], preferred_element_type=jnp.float32)

    s2m_out[...] = (jnp.dot(h_m, wsm2[...], preferred_element_type=jnp.float32)
                    + bm2[...] + col30 * rdeg_m)
    s2u_out[...] = (jnp.dot(h_u, wsu2[...], preferred_element_type=jnp.float32)
                    + bu2[...] + col30 * rdeg_u)


def _dense2_body(s2m, a2m, s2u, a2u, om_out, ou_out):
    s2m_ = s2m[...]
    om_out[...] = s2m_[:, :OUT] + a2m[...][:, :OUT] * s2m_[:, OUT:OUT + 1]
    s2u_ = s2u[...]
    ou_out[...] = s2u_[:, :OUT] + a2u[...][:, :OUT] * s2u_[:, OUT:OUT + 1]


def _row_spec(width):
    return pl.BlockSpec((R, width), lambda i: (i, 0))


def _full_spec(shape):
    return pl.BlockSpec(shape, lambda i: (0, 0))


def _pad_cols(x, width):
    return jnp.pad(x, ((0, 0), (0, width - x.shape[1])))


def kernel(feat_user, feat_movie, edge_src_user, edge_dst_movie,
           W_self1_m, W_neigh1_m, b1_m, W_self1_u, W_neigh1_u, b1_u,
           W_self2_m, W_neigh2_m, b2_m, W_self2_u, W_neigh2_u, b2_u):
    f32 = jnp.float32

    fu_aug = _pad_cols(jnp.concatenate(
        [feat_user, jnp.ones((N_U, 1), f32)], axis=1), W)
    fm_aug = _pad_cols(jnp.concatenate(
        [feat_movie, jnp.ones((N_M, 1), f32)], axis=1), W)

    acc_m, acc_u = _edge_agg(fu_aug, fm_aug, edge_src_user, edge_dst_movie)

    wsm2 = _pad_cols(W_self2_m, W)
    wnm2 = _pad_cols(W_neigh2_m, W)
    wsu2 = _pad_cols(W_self2_u, W)
    wnu2 = _pad_cols(W_neigh2_u, W)
    bm2 = _pad_cols(b2_m[None, :], W)
    bu2 = _pad_cols(b2_u[None, :], W)
    bm1 = b1_m[None, :]
    bu1 = b1_u[None, :]

    p_m, p_u, s2m, s2u = pl.pallas_call(
        _dense1_body,
        grid=(GRID,),
        in_specs=[
            _row_spec(21), _row_spec(W), _row_spec(20), _row_spec(W),
            _full_spec((21, H)), _full_spec((20, H)), _full_spec((1, H)),
            _full_spec((20, H)), _full_spec((21, H)), _full_spec((1, H)),
            _full_spec((H, W)), _full_spec((H, W)), _full_spec((1, W)),
            _full_spec((H, W)), _full_spec((H, W)), _full_spec((1, W)),
        ],
        out_specs=[_row_spec(W)] * 4,
        out_shape=[jax.ShapeDtypeStruct((N_M, W), f32)] * 4,
    )(feat_movie, acc_m, feat_user, acc_u,
      W_self1_m, W_neigh1_m, bm1, W_self1_u, W_neigh1_u, bu1,
      wsm2, wnm2, bm2, wsu2, wnu2, bu2)

    acc2_m, acc2_u = _edge_agg(p_u, p_m, edge_src_user, edge_dst_movie)

    o_m, o_u = pl.pallas_call(
        _dense2_body,
        grid=(GRID,),
        in_specs=[_row_spec(W)] * 4,
        out_specs=[_row_spec(OUT)] * 2,
        out_shape=[jax.ShapeDtypeStruct((N_M, OUT), f32)] * 2,
    )(s2m, acc2_m, s2u, acc2_u)

    return (o_u, o_m)


# speedup vs baseline: 22.3659x; 2.8748x over previous
import functools

import jax
import jax.numpy as jnp
from jax import lax
from jax.experimental import pallas as pl
from jax.experimental.pallas import tpu as pltpu
from jax.experimental.pallas import tpu_sc as plsc

N_U = 50000
N_M = 50000
E = 1600000
H = 40
OUT = 30

W = 32
W1 = 24
CH = 128
NSUB = 16
NPAD = N_U + 8
NCHUNKP = 12544
E_PAD = NCHUNKP * CH
CPT = NCHUNKP // NSUB
NROWCHUNK = -(-N_M // CH)
LAST_ROWS = N_M - (NROWCHUNK - 1) * CH

_sc_mesh = plsc.VectorSubcoreMesh(core_axis_name="c", subcore_axis_name="s")


def _make_edge_agg(width, nb):
    groups = CPT // nb
    assert groups * nb == CPT

    @functools.partial(
        pl.kernel,
        mesh=_sc_mesh,
        compiler_params=pltpu.CompilerParams(use_tc_tiling_on_sc=False),
        out_type=(
            jax.ShapeDtypeStruct((N_M, width), jnp.float32),
            jax.ShapeDtypeStruct((N_U, width), jnp.float32),
        ),
        scratch_types=[
            pltpu.VMEM((nb, CH), jnp.int32),
            pltpu.VMEM((nb, CH), jnp.int32),
        ] + [pltpu.VMEM((CH, width), jnp.float32) for _ in range(nb)]
        + [
            pltpu.VMEM_SHARED((NPAD, width), jnp.float32),
        ] + [pltpu.SemaphoreType.DMA for _ in range(nb + 1)],
    )
    def _edge_agg(tab_u, tab_m, src_hbm, dst_hbm, accm_out, accu_out, *scr):
        idx_a = scr[0]
        idx_b = scr[1]
        rows = list(scr[2:2 + nb])
        acc = scr[2 + nb]
        gsem = list(scr[3 + nb:3 + 2 * nb])
        ssem = scr[3 + 2 * nb]
        r0 = rows[0]
        cid = lax.axis_index("c")
        sid = lax.axis_index("s")

        zv = jnp.zeros((16,), jnp.float32)

        def _zb(r, carry):
            r0[r, pl.ds(0, 16)] = zv
            r0[r, pl.ds(width - 16, 16)] = zv
            return carry

        lax.fori_loop(0, CH, _zb, 0)

        def _zero_chunk(j, carry):
            k = j * NSUB + sid

            @pl.when(k < NROWCHUNK)
            def _():
                pltpu.sync_copy(r0, acc.at[pl.ds(k * CH, CH)])

            return carry

        lax.fori_loop(0, -(-NROWCHUNK // NSUB), _zero_chunk, 0)

        @pl.when(sid == 0)
        def _():
            pltpu.sync_copy(r0.at[pl.ds(0, 8)], acc.at[pl.ds(N_M, 8)])

        plsc.subcore_barrier()

        def _group(gi, si, tab):
            handles = [pltpu.async_copy(tab.at[gi.at[b]], rows[b], gsem[b])
                       for b in range(nb)]
            sc_handles = []
            for b in range(nb):
                handles[b].wait()
                sc_handles.append(
                    pltpu.async_copy(rows[b], acc.at[si.at[b]], ssem,
                                     add=True))
            for h in sc_handles:
                h.wait()

        def _edge_group(g, carry):
            kbase = sid * CPT + g * nb
            pltpu.sync_copy(src_hbm.at[pl.ds(kbase, nb)], idx_a)
            pltpu.sync_copy(dst_hbm.at[pl.ds(kbase, nb)], idx_b)

            @pl.when(cid == 0)
            def _():
                _group(idx_a, idx_b, tab_u)

            @pl.when(cid == 1)
            def _():
                _group(idx_b, idx_a, tab_m)

            return carry

        lax.fori_loop(0, groups, _edge_group, 0)
        plsc.subcore_barrier()

        def _flush_chunk(j, carry):
            k = j * NSUB + sid
            base = k * CH

            @pl.when(k < NROWCHUNK - 1)
            def _():
                @pl.when(cid == 0)
                def _():
                    pltpu.sync_copy(acc.at[pl.ds(base, CH)],
                                    accm_out.at[pl.ds(base, CH)])

                @pl.when(cid == 1)
                def _():
                    pltpu.sync_copy(acc.at[pl.ds(base, CH)],
                                    accu_out.at[pl.ds(base, CH)])

            @pl.when(k == NROWCHUNK - 1)
            def _():
                @pl.when(cid == 0)
                def _():
                    pltpu.sync_copy(acc.at[pl.ds(base, LAST_ROWS)],
                                    accm_out.at[pl.ds(base, LAST_ROWS)])

                @pl.when(cid == 1)
                def _():
                    pltpu.sync_copy(acc.at[pl.ds(base, LAST_ROWS)],
                                    accu_out.at[pl.ds(base, LAST_ROWS)])

            return carry

        lax.fori_loop(0, -(-NROWCHUNK // NSUB), _flush_chunk, 0)

    return _edge_agg


_edge_agg_l1 = _make_edge_agg(W1, 16)
_edge_agg_l2 = _make_edge_agg(W, 7)


R = 2000
GRID = N_M // R


def _dense1_body(fm, am, fu, au, wsm, wnm, bm, wsu, wnu, bu,
                 wsm2, wnm2, bm2, wsu2, wnu2, bu2,
                 pm_out, pu_out, s2m_out, s2u_out):
    col30 = (lax.broadcasted_iota(jnp.int32, (1, W), 1) == OUT).astype(jnp.float32)

    am_ = am[...]
    rdeg_m = 1.0 / jnp.maximum(am_[:, 20:21], 1.0)
    agg_m = am_[:, :20] * rdeg_m
    h_m = jax.nn.relu(
        jnp.dot(fm[...], wsm[...], preferred_element_type=jnp.float32)
        + jnp.dot(agg_m, wnm[...], preferred_element_type=jnp.float32)
        + bm[...])

    au_ = au[...]
    rdeg_u = 1.0 / jnp.maximum(au_[:, 21:22], 1.0)
    agg_u = au_[:, :21] * rdeg_u
    h_u = jax.nn.relu(
        jnp.dot(fu[...], wsu[...], preferred_element_type=jnp.float32)
        + jnp.dot(agg_u, wnu[...], preferred_element_type=jnp.float32)
        + bu[...])

    pu_out[...] = jnp.dot(h_u, wnm2[...], preferred_element_type=jnp.float32)
    pm_out[...] = jnp.dot(h_m, wnu2[...], preferred_element_type=jnp.float32)

    s2m_out[...] = (jnp.dot(h_m, wsm2[...], preferred_element_type=jnp.float32)
                    + bm2[...] + col30 * rdeg_m)
    s2u_out[...] = (jnp.dot(h_u, wsu2[...], preferred_element_type=jnp.float32)
                    + bu2[...] + col30 * rdeg_u)


def _dense2_body(s2m, a2m, s2u, a2u, om_out, ou_out):
    s2m_ = s2m[...]
    om_out[...] = s2m_[:, :OUT] + a2m[...][:, :OUT] * s2m_[:, OUT:OUT + 1]
    s2u_ = s2u[...]
    ou_out[...] = s2u_[:, :OUT] + a2u[...][:, :OUT] * s2u_[:, OUT:OUT + 1]


def _row_spec(width):
    return pl.BlockSpec((R, width), lambda i: (i, 0))


def _full_spec(shape):
    return pl.BlockSpec(shape, lambda i: (0, 0))


def _pad_to(x, rows, cols):
    return jnp.pad(x, ((0, rows - x.shape[0]), (0, cols - x.shape[1])))


def kernel(feat_user, feat_movie, edge_src_user, edge_dst_movie,
           W_self1_m, W_neigh1_m, b1_m, W_self1_u, W_neigh1_u, b1_u,
           W_self2_m, W_neigh2_m, b2_m, W_self2_u, W_neigh2_u, b2_u):
    f32 = jnp.float32

    fu_aug = _pad_to(jnp.concatenate(
        [feat_user, jnp.ones((N_U, 1), f32)], axis=1), NPAD, W1)
    fm_aug = _pad_to(jnp.concatenate(
        [feat_movie, jnp.ones((N_M, 1), f32)], axis=1), NPAD, W1)

    pad = N_U + (jnp.arange(E_PAD - E, dtype=jnp.int32) % 8)
    src2d = jnp.concatenate([edge_src_user, pad]).reshape(NCHUNKP, CH)
    dst2d = jnp.concatenate([edge_dst_movie, pad]).reshape(NCHUNKP, CH)

    acc_m, acc_u = _edge_agg_l1(fu_aug, fm_aug, src2d, dst2d)

    wsm2 = _pad_to(W_self2_m, H, W)
    wnm2 = _pad_to(W_neigh2_m, H, W)
    wsu2 = _pad_to(W_self2_u, H, W)
    wnu2 = _pad_to(W_neigh2_u, H, W)
    bm2 = _pad_to(b2_m[None, :], 1, W)
    bu2 = _pad_to(b2_u[None, :], 1, W)
    bm1 = b1_m[None, :]
    bu1 = b1_u[None, :]

    p_m, p_u, s2m, s2u = pl.pallas_call(
        _dense1_body,
        grid=(GRID,),
        in_specs=[
            _row_spec(21), _row_spec(W1), _row_spec(20), _row_spec(W1),
            _full_spec((21, H)), _full_spec((20, H)), _full_spec((1, H)),
            _full_spec((20, H)), _full_spec((21, H)), _full_spec((1, H)),
            _full_spec((H, W)), _full_spec((H, W)), _full_spec((1, W)),
            _full_spec((H, W)), _full_spec((H, W)), _full_spec((1, W)),
        ],
        out_specs=[_row_spec(W)] * 4,
        out_shape=[jax.ShapeDtypeStruct((NPAD, W), f32)] * 4,
    )(feat_movie, acc_m, feat_user, acc_u,
      W_self1_m, W_neigh1_m, bm1, W_self1_u, W_neigh1_u, bu1,
      wsm2, wnm2, bm2, wsu2, wnu2, bu2)

    acc2_m, acc2_u = _edge_agg_l2(p_u, p_m, src2d, dst2d)

    o_m, o_u = pl.pallas_call(
        _dense2_body,
        grid=(GRID,),
        in_specs=[_row_spec(W)] * 4,
        out_specs=[_row_spec(OUT)] * 2,
        out_shape=[jax.ShapeDtypeStruct((N_M, OUT), f32)] * 2,
    )(s2m, acc2_m, s2u, acc2_u)

    return (o_u, o_m)


# speedup vs baseline: 27.8322x; 1.2444x over previous
import functools

import jax
import jax.numpy as jnp
from jax import lax
from jax.experimental import pallas as pl
from jax.experimental.pallas import tpu as pltpu
from jax.experimental.pallas import tpu_sc as plsc

N_U = 50000
N_M = 50000
E = 1600000
H = 40
OUT = 30

W = 32
W1 = 24
CH = 128
NSUB = 16
NPAD = N_U + 8
NCHUNKP = 12544
E_PAD = NCHUNKP * CH
CPT = NCHUNKP // NSUB
NROWCHUNK = -(-N_M // CH)
LAST_ROWS = N_M - (NROWCHUNK - 1) * CH

_sc_mesh = plsc.VectorSubcoreMesh(core_axis_name="c", subcore_axis_name="s")


def _make_edge_agg(width, nb):
    groups = CPT // nb
    sgroups = groups // 2
    assert groups * nb == CPT and sgroups * 2 == groups

    @functools.partial(
        pl.kernel,
        mesh=_sc_mesh,
        compiler_params=pltpu.CompilerParams(use_tc_tiling_on_sc=False),
        out_type=(
            jax.ShapeDtypeStruct((N_M, width), jnp.float32),
            jax.ShapeDtypeStruct((N_U, width), jnp.float32),
        ),
        scratch_types=[
            pltpu.VMEM((nb, CH), jnp.int32) for _ in range(4)
        ] + [pltpu.VMEM((CH, width), jnp.float32) for _ in range(nb)]
        + [
            pltpu.VMEM_SHARED((NPAD, width), jnp.float32),
        ] + [pltpu.SemaphoreType.DMA for _ in range(2 * nb + 2)],
    )
    def _edge_agg(tab_u, tab_m, src_hbm, dst_hbm, accm_out, accu_out, *scr):
        sA, dA, sB, dB = scr[0], scr[1], scr[2], scr[3]
        rows = list(scr[4:4 + nb])
        acc = scr[4 + nb]
        gsem = list(scr[5 + nb:5 + 2 * nb])
        ssem = list(scr[5 + 2 * nb:5 + 3 * nb])
        isem = scr[5 + 3 * nb]
        isem2 = scr[6 + 3 * nb]
        r0 = rows[0]
        cid = lax.axis_index("c")
        sid = lax.axis_index("s")

        zv = jnp.zeros((16,), jnp.float32)

        def _zb(r, carry):
            r0[r, pl.ds(0, 16)] = zv
            r0[r, pl.ds(width - 16, 16)] = zv
            return carry

        lax.fori_loop(0, CH, _zb, 0)

        def _zero_chunk(j, carry):
            k = j * NSUB + sid

            @pl.when(k < NROWCHUNK)
            def _():
                pltpu.sync_copy(r0, acc.at[pl.ds(k * CH, CH)])

            return carry

        lax.fori_loop(0, -(-NROWCHUNK // NSUB), _zero_chunk, 0)

        @pl.when(sid == 0)
        def _():
            pltpu.sync_copy(r0.at[pl.ds(0, 8)], acc.at[pl.ds(N_M, 8)])

        plsc.subcore_barrier()

        def _drain_scatter(b):
            pltpu.make_async_copy(accm_out.at[pl.ds(0, CH)], rows[b],
                                  ssem[b]).wait()

        def _wait_idx(bs, bd, sem):
            pltpu.make_async_copy(src_hbm.at[pl.ds(0, nb)], bs, sem).wait()
            pltpu.make_async_copy(src_hbm.at[pl.ds(0, nb)], bd, sem).wait()

        def _half(gi, si, tab, guard, prefetch):
            handles = []
            for b in range(nb):
                if guard is None:
                    _drain_scatter(b)
                else:
                    @pl.when(guard)
                    def _(b=b):
                        _drain_scatter(b)
                handles.append(
                    pltpu.async_copy(tab.at[gi.at[b]], rows[b], gsem[b]))
            prefetch()
            for b in range(nb):
                handles[b].wait()
                pltpu.async_copy(rows[b], acc.at[si.at[b]], ssem[b],
                                 add=True)

        pltpu.sync_copy(src_hbm.at[pl.ds(sid * CPT, nb)], sA)
        pltpu.sync_copy(dst_hbm.at[pl.ds(sid * CPT, nb)], dA)

        def _sgroup(sg, carry):
            kbase = sid * CPT + sg * (2 * nb)

            def _pf_b():
                pltpu.async_copy(src_hbm.at[pl.ds(kbase + nb, nb)], sB, isem)
                pltpu.async_copy(dst_hbm.at[pl.ds(kbase + nb, nb)], dB, isem)

            @pl.when(cid == 0)
            def _():
                _half(sA, dA, tab_u, sg > 0, _pf_b)

            @pl.when(cid == 1)
            def _():
                _half(dA, sA, tab_m, sg > 0, _pf_b)

            _wait_idx(sB, dB, isem)

            def _pf_a():
                @pl.when(sg < sgroups - 1)
                def _():
                    pltpu.async_copy(src_hbm.at[pl.ds(kbase + 2 * nb, nb)],
                                     sA, isem2)
                    pltpu.async_copy(dst_hbm.at[pl.ds(kbase + 2 * nb, nb)],
                                     dA, isem2)

            @pl.when(cid == 0)
            def _():
                _half(sB, dB, tab_u, None, _pf_a)

            @pl.when(cid == 1)
            def _():
                _half(dB, sB, tab_m, None, _pf_a)

            @pl.when(sg < sgroups - 1)
            def _():
                _wait_idx(sA, dA, isem2)

            return carry

        lax.fori_loop(0, sgroups, _sgroup, 0)
        for b in range(nb):
            _drain_scatter(b)
        plsc.subcore_barrier()

        def _flush_chunk(j, carry):
            k = j * NSUB + sid
            base = k * CH

            @pl.when(k < NROWCHUNK - 1)
            def _():
                @pl.when(cid == 0)
                def _():
                    pltpu.sync_copy(acc.at[pl.ds(base, CH)],
                                    accm_out.at[pl.ds(base, CH)])

                @pl.when(cid == 1)
                def _():
                    pltpu.sync_copy(acc.at[pl.ds(base, CH)],
                                    accu_out.at[pl.ds(base, CH)])

            @pl.when(k == NROWCHUNK - 1)
            def _():
                @pl.when(cid == 0)
                def _():
                    pltpu.sync_copy(acc.at[pl.ds(base, LAST_ROWS)],
                                    accm_out.at[pl.ds(base, LAST_ROWS)])

                @pl.when(cid == 1)
                def _():
                    pltpu.sync_copy(acc.at[pl.ds(base, LAST_ROWS)],
                                    accu_out.at[pl.ds(base, LAST_ROWS)])

            return carry

        lax.fori_loop(0, -(-NROWCHUNK // NSUB), _flush_chunk, 0)

    return _edge_agg


_edge_agg_l1 = _make_edge_agg(W1, 14)
_edge_agg_l2 = _make_edge_agg(W, 4)


R = 2000
GRID = N_M // R


def _dense1_body(fm, am, fu, au, wsm, wnm, bm, wsu, wnu, bu,
                 wsm2, wnm2, bm2, wsu2, wnu2, bu2,
                 pm_out, pu_out, s2m_out, s2u_out):
    col30 = (lax.broadcasted_iota(jnp.int32, (1, W), 1) == OUT).astype(jnp.float32)

    am_ = am[...]
    rdeg_m = 1.0 / jnp.maximum(am_[:, 20:21], 1.0)
    agg_m = am_[:, :20] * rdeg_m
    h_m = jax.nn.relu(
        jnp.dot(fm[...], wsm[...], preferred_element_type=jnp.float32)
        + jnp.dot(agg_m, wnm[...], preferred_element_type=jnp.float32)
        + bm[...])

    au_ = au[...]
    rdeg_u = 1.0 / jnp.maximum(au_[:, 21:22], 1.0)
    agg_u = au_[:, :21] * rdeg_u
    h_u = jax.nn.relu(
        jnp.dot(fu[...], wsu[...], preferred_element_type=jnp.float32)
        + jnp.dot(agg_u, wnu[...], preferred_element_type=jnp.float32)
        + bu[...])

    pu_out[...] = jnp.dot(h_u, wnm2[...], preferred_element_type=jnp.float32)
    pm_out[...] = jnp.dot(h_m, wnu2[...], preferred_element_type=jnp.float32)

    s2m_out[...] = (jnp.dot(h_m, wsm2[...], preferred_element_type=jnp.float32)
                    + bm2[...] + col30 * rdeg_m)
    s2u_out[...] = (jnp.dot(h_u, wsu2[...], preferred_element_type=jnp.float32)
                    + bu2[...] + col30 * rdeg_u)


def _dense2_body(s2m, a2m, s2u, a2u, om_out, ou_out):
    s2m_ = s2m[...]
    om_out[...] = s2m_[:, :OUT] + a2m[...][:, :OUT] * s2m_[:, OUT:OUT + 1]
    s2u_ = s2u[...]
    ou_out[...] = s2u_[:, :OUT] + a2u[...][:, :OUT] * s2u_[:, OUT:OUT + 1]


def _row_spec(width):
    return pl.BlockSpec((R, width), lambda i: (i, 0))


def _full_spec(shape):
    return pl.BlockSpec(shape, lambda i: (0, 0))


def _pad_to(x, rows, cols):
    return jnp.pad(x, ((0, rows - x.shape[0]), (0, cols - x.shape[1])))


def kernel(feat_user, feat_movie, edge_src_user, edge_dst_movie,
           W_self1_m, W_neigh1_m, b1_m, W_self1_u, W_neigh1_u, b1_u,
           W_self2_m, W_neigh2_m, b2_m, W_self2_u, W_neigh2_u, b2_u):
    f32 = jnp.float32

    fu_aug = _pad_to(jnp.concatenate(
        [feat_user, jnp.ones((N_U, 1), f32)], axis=1), NPAD, W1)
    fm_aug = _pad_to(jnp.concatenate(
        [feat_movie, jnp.ones((N_M, 1), f32)], axis=1), NPAD, W1)

    pad = N_U + (jnp.arange(E_PAD - E, dtype=jnp.int32) % 8)
    src2d = jnp.concatenate([edge_src_user, pad]).reshape(NCHUNKP, CH)
    dst2d = jnp.concatenate([edge_dst_movie, pad]).reshape(NCHUNKP, CH)

    acc_m, acc_u = _edge_agg_l1(fu_aug, fm_aug, src2d, dst2d)

    wsm2 = _pad_to(W_self2_m, H, W)
    wnm2 = _pad_to(W_neigh2_m, H, W)
    wsu2 = _pad_to(W_self2_u, H, W)
    wnu2 = _pad_to(W_neigh2_u, H, W)
    bm2 = _pad_to(b2_m[None, :], 1, W)
    bu2 = _pad_to(b2_u[None, :], 1, W)
    bm1 = b1_m[None, :]
    bu1 = b1_u[None, :]

    p_m, p_u, s2m, s2u = pl.pallas_call(
        _dense1_body,
        grid=(GRID,),
        in_specs=[
            _row_spec(21), _row_spec(W1), _row_spec(20), _row_spec(W1),
            _full_spec((21, H)), _full_spec((20, H)), _full_spec((1, H)),
            _full_spec((20, H)), _full_spec((21, H)), _full_spec((1, H)),
            _full_spec((H, W)), _full_spec((H, W)), _full_spec((1, W)),
            _full_spec((H, W)), _full_spec((H, W)), _full_spec((1, W)),
        ],
        out_specs=[_row_spec(W)] * 4,
        out_shape=[jax.ShapeDtypeStruct((NPAD, W), f32)] * 4,
    )(feat_movie, acc_m, feat_user, acc_u,
      W_self1_m, W_neigh1_m, bm1, W_self1_u, W_neigh1_u, bu1,
      wsm2, wnm2, bm2, wsu2, wnu2, bu2)

    acc2_m, acc2_u = _edge_agg_l2(p_u, p_m, src2d, dst2d)

    o_m, o_u = pl.pallas_call(
        _dense2_body,
        grid=(GRID,),
        in_specs=[_row_spec(W)] * 4,
        out_specs=[_row_spec(OUT)] * 2,
        out_shape=[jax.ShapeDtypeStruct((N_M, OUT), f32)] * 2,
    )(s2m, acc2_m, s2u, acc2_u)

    return (o_u, o_m)


# speedup vs baseline: 28.7953x; 1.0346x over previous
import functools

import jax
import jax.numpy as jnp
from jax import lax
from jax.experimental import pallas as pl
from jax.experimental.pallas import tpu as pltpu
from jax.experimental.pallas import tpu_sc as plsc

N_U = 50000
N_M = 50000
E = 1600000
H = 40
OUT = 30

W = 32
W1 = 24
CH = 125
NSUB = 16
NCHUNK = E // CH
CPT = NCHUNK // NSUB
NROWCHUNK = N_M // CH
RPT = NROWCHUNK // NSUB

_sc_mesh = plsc.VectorSubcoreMesh(core_axis_name="c", subcore_axis_name="s")


def _make_edge_agg(width, nb):
    groups = CPT // nb
    sgroups = groups // 2
    assert groups * nb == CPT and sgroups * 2 == groups

    @functools.partial(
        pl.kernel,
        mesh=_sc_mesh,
        compiler_params=pltpu.CompilerParams(use_tc_tiling_on_sc=False),
        out_type=(
            jax.ShapeDtypeStruct((N_M, width), jnp.float32),
            jax.ShapeDtypeStruct((N_U, width), jnp.float32),
        ),
        scratch_types=[
            pltpu.VMEM((nb, CH), jnp.int32) for _ in range(4)
        ] + [pltpu.VMEM((CH, width), jnp.float32) for _ in range(nb)]
        + [
            pltpu.VMEM_SHARED((N_M, width), jnp.float32),
        ] + [pltpu.SemaphoreType.DMA for _ in range(2 * nb + 2)],
    )
    def _edge_agg(tab_u, tab_m, src_hbm, dst_hbm, accm_out, accu_out, *scr):
        sA, dA, sB, dB = scr[0], scr[1], scr[2], scr[3]
        rows = list(scr[4:4 + nb])
        acc = scr[4 + nb]
        gsem = list(scr[5 + nb:5 + 2 * nb])
        ssem = list(scr[5 + 2 * nb:5 + 3 * nb])
        isem = scr[5 + 3 * nb]
        isem2 = scr[6 + 3 * nb]
        r0 = rows[0]
        cid = lax.axis_index("c")
        sid = lax.axis_index("s")

        zv = jnp.zeros((16,), jnp.float32)

        def _zb(r, carry):
            r0[r, pl.ds(0, 16)] = zv
            r0[r, pl.ds(width - 16, 16)] = zv
            return carry

        lax.fori_loop(0, CH, _zb, 0)

        def _zero_chunk(j, carry):
            k = j * NSUB + sid
            pltpu.sync_copy(r0, acc.at[pl.ds(k * CH, CH)])
            return carry

        lax.fori_loop(0, RPT, _zero_chunk, 0)
        plsc.subcore_barrier()

        def _drain_scatter(b):
            pltpu.make_async_copy(accm_out.at[pl.ds(0, CH)], rows[b],
                                  ssem[b]).wait()

        def _wait_idx(bs, bd, sem):
            pltpu.make_async_copy(src_hbm.at[pl.ds(0, nb)], bs, sem).wait()
            pltpu.make_async_copy(src_hbm.at[pl.ds(0, nb)], bd, sem).wait()

        def _half(gi, si, tab, guard, prefetch):
            handles = []
            for b in range(nb):
                if guard is None:
                    _drain_scatter(b)
                else:
                    @pl.when(guard)
                    def _(b=b):
                        _drain_scatter(b)
                handles.append(
                    pltpu.async_copy(tab.at[gi.at[b]], rows[b], gsem[b]))
            prefetch()
            for b in range(nb):
                handles[b].wait()
                pltpu.async_copy(rows[b], acc.at[si.at[b]], ssem[b],
                                 add=True)

        pltpu.sync_copy(src_hbm.at[pl.ds(sid * CPT, nb)], sA)
        pltpu.sync_copy(dst_hbm.at[pl.ds(sid * CPT, nb)], dA)

        def _sgroup(sg, carry):
            kbase = sid * CPT + sg * (2 * nb)

            def _pf_b():
                pltpu.async_copy(src_hbm.at[pl.ds(kbase + nb, nb)], sB, isem)
                pltpu.async_copy(dst_hbm.at[pl.ds(kbase + nb, nb)], dB, isem)

            @pl.when(cid == 0)
            def _():
                _half(sA, dA, tab_u, sg > 0, _pf_b)

            @pl.when(cid == 1)
            def _():
                _half(dA, sA, tab_m, sg > 0, _pf_b)

            _wait_idx(sB, dB, isem)

            def _pf_a():
                @pl.when(sg < sgroups - 1)
                def _():
                    pltpu.async_copy(src_hbm.at[pl.ds(kbase + 2 * nb, nb)],
                                     sA, isem2)
                    pltpu.async_copy(dst_hbm.at[pl.ds(kbase + 2 * nb, nb)],
                                     dA, isem2)

            @pl.when(cid == 0)
            def _():
                _half(sB, dB, tab_u, None, _pf_a)

            @pl.when(cid == 1)
            def _():
                _half(dB, sB, tab_m, None, _pf_a)

            @pl.when(sg < sgroups - 1)
            def _():
                _wait_idx(sA, dA, isem2)

            return carry

        lax.fori_loop(0, sgroups, _sgroup, 0)
        for b in range(nb):
            _drain_scatter(b)
        plsc.subcore_barrier()

        def _flush_chunk(j, carry):
            base = (j * NSUB + sid) * CH

            @pl.when(cid == 0)
            def _():
                pltpu.sync_copy(acc.at[pl.ds(base, CH)],
                                accm_out.at[pl.ds(base, CH)])

            @pl.when(cid == 1)
            def _():
                pltpu.sync_copy(acc.at[pl.ds(base, CH)],
                                accu_out.at[pl.ds(base, CH)])

            return carry

        lax.fori_loop(0, RPT, _flush_chunk, 0)

    return _edge_agg


_edge_agg_l1 = _make_edge_agg(W1, 10)
_edge_agg_l2 = _make_edge_agg(W, 5)


R = 5000
GRID = N_M // R


def _dense1_body(fm, am, fu, au, wsm, wnm, bm, wsu, wnu, bu,
                 wsm2, wnm2, bm2, wsu2, wnu2, bu2,
                 pm_out, pu_out, s2m_out, s2u_out):
    col30 = (lax.broadcasted_iota(jnp.int32, (1, W), 1) == OUT).astype(jnp.float32)

    am_ = am[...]
    rdeg_m = 1.0 / jnp.maximum(am_[:, 20:21], 1.0)
    agg_m = am_[:, :20] * rdeg_m
    h_m = jax.nn.relu(
        jnp.dot(fm[...], wsm[...], preferred_element_type=jnp.float32)
        + jnp.dot(agg_m, wnm[...], preferred_element_type=jnp.float32)
        + bm[...])

    au_ = au[...]
    rdeg_u = 1.0 / jnp.maximum(au_[:, 21:22], 1.0)
    agg_u = au_[:, :21] * rdeg_u
    h_u = jax.nn.relu(
        jnp.dot(fu[...], wsu[...], preferred_element_type=jnp.float32)
        + jnp.dot(agg_u, wnu[...], preferred_element_type=jnp.float32)
        + bu[...])

    pu_out[...] = jnp.dot(h_u, wnm2[...], preferred_element_type=jnp.float32)
    pm_out[...] = jnp.dot(h_m, wnu2[...], preferred_element_type=jnp.float32)

    s2m_out[...] = (jnp.dot(h_m, wsm2[...], preferred_element_type=jnp.float32)
                    + bm2[...] + col30 * rdeg_m)
    s2u_out[...] = (jnp.dot(h_u, wsu2[...], preferred_element_type=jnp.float32)
                    + bu2[...] + col30 * rdeg_u)


def _dense2_body(s2m, a2m, s2u, a2u, om_out, ou_out):
    s2m_ = s2m[...]
    om_out[...] = s2m_[:, :OUT] + a2m[...][:, :OUT] * s2m_[:, OUT:OUT + 1]
    s2u_ = s2u[...]
    ou_out[...] = s2u_[:, :OUT] + a2u[...][:, :OUT] * s2u_[:, OUT:OUT + 1]


def _row_spec(width):
    return pl.BlockSpec((R, width), lambda i: (i, 0))


def _full_spec(shape):
    return pl.BlockSpec(shape, lambda i: (0, 0))


def _pad_to(x, rows, cols):
    return jnp.pad(x, ((0, rows - x.shape[0]), (0, cols - x.shape[1])))


def kernel(feat_user, feat_movie, edge_src_user, edge_dst_movie,
           W_self1_m, W_neigh1_m, b1_m, W_self1_u, W_neigh1_u, b1_u,
           W_self2_m, W_neigh2_m, b2_m, W_self2_u, W_neigh2_u, b2_u):
    f32 = jnp.float32

    fu_aug = _pad_to(jnp.concatenate(
        [feat_user, jnp.ones((N_U, 1), f32)], axis=1), N_U, W1)
    fm_aug = _pad_to(jnp.concatenate(
        [feat_movie, jnp.ones((N_M, 1), f32)], axis=1), N_M, W1)

    src2d = edge_src_user.reshape(NCHUNK, CH)
    dst2d = edge_dst_movie.reshape(NCHUNK, CH)

    acc_m, acc_u = _edge_agg_l1(fu_aug, fm_aug, src2d, dst2d)

    wsm2 = _pad_to(W_self2_m, H, W)
    wnm2 = _pad_to(W_neigh2_m, H, W)
    wsu2 = _pad_to(W_self2_u, H, W)
    wnu2 = _pad_to(W_neigh2_u, H, W)
    bm2 = _pad_to(b2_m[None, :], 1, W)
    bu2 = _pad_to(b2_u[None, :], 1, W)
    bm1 = b1_m[None, :]
    bu1 = b1_u[None, :]

    p_m, p_u, s2m, s2u = pl.pallas_call(
        _dense1_body,
        grid=(GRID,),
        in_specs=[
            _row_spec(21), _row_spec(W1), _row_spec(20), _row_spec(W1),
            _full_spec((21, H)), _full_spec((20, H)), _full_spec((1, H)),
            _full_spec((20, H)), _full_spec((21, H)), _full_spec((1, H)),
            _full_spec((H, W)), _full_spec((H, W)), _full_spec((1, W)),
            _full_spec((H, W)), _full_spec((H, W)), _full_spec((1, W)),
        ],
        out_specs=[_row_spec(W)] * 4,
        out_shape=[jax.ShapeDtypeStruct((N_M, W), f32)] * 4,
    )(feat_movie, acc_m, feat_user, acc_u,
      W_self1_m, W_neigh1_m, bm1, W_self1_u, W_neigh1_u, bu1,
      wsm2, wnm2, bm2, wsu2, wnu2, bu2)

    acc2_m, acc2_u = _edge_agg_l2(p_u, p_m, src2d, dst2d)

    o_m, o_u = pl.pallas_call(
        _dense2_body,
        grid=(GRID,),
        in_specs=[_row_spec(W)] * 4,
        out_specs=[_row_spec(OUT)] * 2,
        out_shape=[jax.ShapeDtypeStruct((N_M, OUT), f32)] * 2,
    )(s2m, acc2_m, s2u, acc2_u)

    return (o_u, o_m)
